# baseline (speedup 1.0000x reference)
import functools

import jax
import jax.numpy as jnp
from jax import lax
from jax.experimental import pallas as pl
from jax.experimental.pallas import tpu as pltpu
from jax.experimental.pallas import tpu_sc as plsc

_N_NODES = 10000
_N_EDGES = 320000
_H = 128
_BATCH = 2

_NC = 2
_NS = 16
_L = 16

_EC = 64
_K = 16
_NBUF = 4
_EDGES_PER_TILE = _N_EDGES // _NS
_NCHUNK = 320
_NSUP = _NCHUNK // _K
_EPAD = _NCHUNK * _EC
_WB = 40
_WB_TOTAL = _N_NODES // _WB
_WB_PER_TILE = (_WB_TOTAL + _NS - 1) // _NS

_mesh = plsc.VectorSubcoreMesh(core_axis_name="c", subcore_axis_name="s")


@functools.partial(
    pl.kernel,
    out_type=jax.ShapeDtypeStruct((_BATCH, _N_NODES, _H), jnp.float32),
    mesh=_mesh,
    scratch_types=[
        pltpu.VMEM_SHARED((_N_NODES, _H), jnp.float32),
        pltpu.VMEM((2, _K, _EC), jnp.int32),
        pltpu.VMEM((2, _K, _EC), jnp.int32),
        pltpu.VMEM((2, _K, _EC), jnp.float32),
        pltpu.VMEM((_NBUF, _EC, _H), jnp.float32),
        pltpu.SemaphoreType.DMA,
        pltpu.SemaphoreType.DMA,
        pltpu.SemaphoreType.DMA,
        pltpu.SemaphoreType.DMA,
        pltpu.SemaphoreType.DMA,
        pltpu.SemaphoreType.DMA,
    ],
)
def _neighbor_agg(h_ref, n1_ref, n2_ref, w_ref, out_ref,
                  acc, idx1s, idx2s, wvs, rbs, gs0, gs1, gs2, gs3, is0, is1):
    c = lax.axis_index("c")
    s = lax.axis_index("s")
    gsems = (gs0, gs1, gs2, gs3)

    def sup_copies(k, par):
        isem = is0 if par == 0 else is1
        src = lambda ref: ref.at[c].at[s].at[pl.ds(k * _K, _K)]
        return (
            pltpu.make_async_copy(src(n1_ref), idx1s.at[par], isem),
            pltpu.make_async_copy(src(n2_ref), idx2s.at[par], isem),
            pltpu.make_async_copy(src(w_ref), wvs.at[par], isem),
        )

    def sup_issue(k, par):
        for cp in sup_copies(k, par):
            cp.start()

    def sup_wait(k, par):
        for cp in sup_copies(k, par):
            cp.wait()

    def gather(idx_row, b):
        pltpu.async_copy(h_ref.at[idx_row], rbs.at[b], gsems[b])

    def gather_wait(idx_row, b):
        pltpu.make_async_copy(h_ref.at[idx_row], rbs.at[b], gsems[b]).wait()

    zero = jnp.zeros((_L,), jnp.float32)
    zb = rbs.at[0]

    def zrow(r, carry):
        for f in range(_H // _L):
            zb[r, pl.ds(f * _L, _L)] = zero
        return carry

    lax.fori_loop(0, _WB, zrow, 0)
    for k in range(_WB_PER_TILE):
        m = s + _NS * k

        @pl.when(m < _WB_TOTAL)
        def _():
            pltpu.sync_copy(zb.at[pl.ds(0, _WB)], acc.at[pl.ds(m * _WB, _WB)])

    plsc.subcore_barrier()

    sup_issue(0, 0)
    sup_wait(0, 0)
    for b in range(_NBUF - 1):
        gather(idx2s.at[0].at[b], b)

    def scale(wvp, j, rb):
        def mgroup(g, carry):
            w16 = wvp[j, pl.ds(g * _L, _L)]
            for jj in range(_L):
                ws = w16[jj]
                e = g * _L + jj
                for f in range(_H // _L):
                    sl = pl.ds(f * _L, _L)
                    rb[e, sl] = rb[e, sl] * ws
            return carry

        lax.fori_loop(0, _EC // _L, mgroup, 0)

    def outer(ksup2, carry):
        for par in (0, 1):
            ksup = ksup2 * 2 + par
            parn = 1 - par
            idx1p = idx1s.at[par]
            idx2p = idx2s.at[par]
            wvp = wvs.at[par]

            @pl.when(ksup < _NSUP - 1)
            def _():
                sup_issue(ksup + 1, parn)

            def inner(j2, carry2):
                for b in range(_NBUF):
                    jj = j2 * _NBUF + b
                    bn = (b + _NBUF - 1) % _NBUF
                    last_j2 = j2 == _K // _NBUF - 1
                    if b == 0:
                        pltpu.async_copy(
                            h_ref.at[idx2p.at[jj + 3]], rbs.at[bn], gsems[bn])
                    else:
                        @pl.when(jnp.logical_not(last_j2))
                        def _():
                            pltpu.async_copy(
                                h_ref.at[idx2p.at[jj + 3]],
                                rbs.at[bn], gsems[bn])

                        @pl.when(last_j2 & (ksup < _NSUP - 1))
                        def _():
                            if b == 1:
                                sup_wait(ksup + 1, parn)
                            pltpu.async_copy(
                                h_ref.at[idx2s.at[parn].at[b - 1]],
                                rbs.at[bn], gsems[bn])

                    gather_wait(idx2p.at[jj], b)
                    scale(wvp, jj, rbs.at[b])
                    pltpu.sync_copy(rbs.at[b], acc.at[idx1p.at[jj]], add=True)
                return carry2

            lax.fori_loop(0, _K // _NBUF, inner, 0)
        return carry

    lax.fori_loop(0, _NSUP // 2, outer, 0)
    plsc.subcore_barrier()

    wb = rbs.at[1]
    for k in range(_WB_PER_TILE):
        m = s + _NS * k

        @pl.when(m < _WB_TOTAL)
        def _():
            pltpu.sync_copy(acc.at[pl.ds(m * _WB, _WB)], wb.at[pl.ds(0, _WB)])
            pltpu.sync_copy(wb.at[pl.ds(0, _WB)],
                            out_ref.at[c, pl.ds(m * _WB, _WB)])


def kernel(H, edge_weights):
    n1 = edge_weights[..., 0].astype(jnp.int32)
    n2 = edge_weights[..., 1].astype(jnp.int32)
    w = edge_weights[..., 2]
    offs = (jnp.arange(_BATCH, dtype=jnp.int32) * _N_NODES)[:, None]
    n2g = n2 + offs

    pad = _EPAD - _EDGES_PER_TILE

    def chunked(x):
        x = x.reshape(_BATCH, _NS, _EDGES_PER_TILE)
        x = jnp.pad(x, ((0, 0), (0, 0), (0, pad)))
        return x.reshape(_BATCH, _NS, _NCHUNK, _EC)

    h_flat = H.reshape(_BATCH * _N_NODES, _H)
    return _neighbor_agg(h_flat, chunked(n1), chunked(n2g), chunked(w))
